# baseline (speedup 1.0000x reference)
import jax
import jax.numpy as jnp
from jax.experimental import pallas as pl

_NSTREAM = 4
_BM = 512


def _router_kernel(*refs):
    x_refs = refs[:_NSTREAM]
    w_ref = refs[_NSTREAM]
    o_ref = refs[_NSTREAM + 1]
    w = w_ref[...].astype(jnp.bfloat16)
    for j, x_ref in enumerate(x_refs):
        logits = jax.lax.dot_general(
            x_ref[...].astype(jnp.bfloat16), w,
            dimension_numbers=(((1,), (1,)), ((), ())),
            preferred_element_type=jnp.float32,
        )
        o_ref[j * _BM:(j + 1) * _BM, :] = jnp.maximum(logits, 0.0)


def kernel(x, W):
    M, K = x.shape
    E = W.shape[0]
    rows_per_step = _NSTREAM * _BM
    in_specs = [
        pl.BlockSpec((_BM, K), lambda i, j=j: (i * _NSTREAM + j, 0))
        for j in range(_NSTREAM)
    ] + [pl.BlockSpec((E, K), lambda i: (0, 0))]
    return pl.pallas_call(
        _router_kernel,
        grid=(M // rows_per_step,),
        in_specs=in_specs,
        out_specs=pl.BlockSpec((rows_per_step, E), lambda i: (i, 0)),
        out_shape=jax.ShapeDtypeStruct((M, E), x.dtype),
    )(*([x] * _NSTREAM), W)
